# baseline (speedup 1.0000x reference)
import jax
import jax.numpy as jnp
from jax import lax
from jax.experimental import pallas as pl
from jax.experimental.pallas import tpu as pltpu
from jax.experimental.pallas import tpu_sc as plsc

B = 16384
L = 200
D = 64
N = 1_000_000
LP = 208
NEG = -1e9

_BK = 8192


def _score_table_body(ntt_ref, w1_ref, b1_ref, w2_ref, b2_ref, out_ref):
    i = pl.program_id(0)
    x = ntt_ref[...]
    a = jnp.tanh(
        lax.dot_general(w1_ref[...], x, (((1,), (0,)), ((), ())),
                        preferred_element_type=jnp.float32)
        + b1_ref[...][:, None])
    g = lax.dot_general(w2_ref[...], a, (((1,), (0,)), ((), ())),
                        preferred_element_type=jnp.float32)
    g = (g + b2_ref[...][None, :]).reshape(_BK)
    nidx = lax.broadcasted_iota(jnp.int32, (_BK,), 0) + i * _BK
    out_ref[...] = jnp.where(nidx == 0, jnp.float32(NEG), g)


def _score_table(news_t, W_a1, b_a1, W_a2, b_a2):
    grid = pl.cdiv(N, _BK)
    return pl.pallas_call(
        _score_table_body,
        grid=(grid,),
        in_specs=[
            pl.BlockSpec((D, _BK), lambda i: (0, i)),
            pl.BlockSpec((D // 2, D), lambda i: (0, 0)),
            pl.BlockSpec((D // 2,), lambda i: (0,)),
            pl.BlockSpec((1, D // 2), lambda i: (0, 0)),
            pl.BlockSpec((1,), lambda i: (0,)),
        ],
        out_specs=pl.BlockSpec((_BK,), lambda i: (i,)),
        out_shape=jax.ShapeDtypeStruct((N,), jnp.float32),
    )(news_t, W_a1, b_a1, W_a2, b_a2)


def _sc_body(hist_hbm, g_hbm, news_hbm, nidx_hbm, uidx_hbm, utab_hbm,
             hnum_hbm, den_hbm, iemb_hbm, uemb_hbm,
             hist_v, s2_v, w_v, rows_v, out_v, den_v, ie_v, idx_v,
             sem_s0, sem_s1, sem_r0, sem_r1, sem_h, sem_i):
    nc = 2
    wid = lax.axis_index("s") * nc + lax.axis_index("c")
    bpw = B // 32
    b0 = wid * bpw
    cc = 16
    nchunk = bpw // cc
    sem_s = (sem_s0, sem_s1)
    sem_r = (sem_r0, sem_r1)

    zf = jnp.zeros((16,), jnp.float32)
    for hh in range(2):
        for r in range(cc):
            s2_v[hh, r, pl.ds(192, 16)] = jnp.full((16,), NEG, jnp.float32)
    for nb in range(2):
        for r in range(L, LP):
            for q in range(D // 16):
                rows_v[nb, r, pl.ds(16 * q, 16)] = zf

    pltpu.sync_copy(uidx_hbm.at[pl.ds(b0, bpw)], idx_v)
    u_descs = [
        pltpu.async_copy(utab_hbm.at[idx_v.at[pl.ds(j * 128, 128)]],
                         out_v.at[pl.ds(j * 128, 128), :], sem_i)
        for j in range(bpw // 128)
    ]
    for c in u_descs:
        c.wait()
    pltpu.sync_copy(out_v, uemb_hbm.at[pl.ds(b0, bpw)])
    pltpu.sync_copy(nidx_hbm.at[pl.ds(b0, bpw)], idx_v)
    id_descs = [
        pltpu.async_copy(news_hbm.at[idx_v.at[pl.ds(j * 128, 128)]],
                         ie_v.at[pl.ds(j * 128, 128), :], sem_i)
        for j in range(bpw // 128)
    ]

    def _issue_scores(hh, r):
        pltpu.async_copy(g_hbm.at[hist_v.at[hh, r, pl.ds(0, 104)]],
                         s2_v.at[hh, r, pl.ds(0, 104)], sem_s[hh])
        pltpu.async_copy(g_hbm.at[hist_v.at[hh, r, pl.ds(104, 96)]],
                         s2_v.at[hh, r, pl.ds(104, 96)], sem_s[hh])

    def _drain_scores(hh, r):
        pltpu.make_async_copy(g_hbm.at[pl.ds(0, 104)],
                              s2_v.at[hh, r, pl.ds(0, 104)], sem_s[hh]).wait()
        pltpu.make_async_copy(g_hbm.at[pl.ds(0, 96)],
                              s2_v.at[hh, r, pl.ds(104, 96)], sem_s[hh]).wait()

    def _issue_rows(hh, r, nb):
        pltpu.async_copy(news_hbm.at[hist_v.at[hh, r, pl.ds(0, 104)]],
                         rows_v.at[nb, pl.ds(0, 104), :], sem_r[nb])
        pltpu.async_copy(news_hbm.at[hist_v.at[hh, r, pl.ds(104, 96)]],
                         rows_v.at[nb, pl.ds(104, 96), :], sem_r[nb])

    def _drain_rows(nb):
        pltpu.make_async_copy(news_hbm.at[pl.ds(0, 104), :],
                              rows_v.at[nb, pl.ds(0, 104), :],
                              sem_r[nb]).wait()
        pltpu.make_async_copy(news_hbm.at[pl.ds(0, 96), :],
                              rows_v.at[nb, pl.ds(104, 96), :],
                              sem_r[nb]).wait()

    pltpu.async_copy(hist_hbm.at[pl.ds(b0, cc), :], hist_v.at[0],
                     sem_h).wait()
    for r in range(cc):
        _issue_scores(0, r)

    @pl.loop(0, nchunk // 2)
    def _pair(i):
        for hh in range(2):
            ci = 2 * i + hh

            @pl.when(ci + 1 < nchunk)
            def _():
                pltpu.async_copy(
                    hist_hbm.at[pl.ds(b0 + (ci + 1) * cc, cc), :],
                    hist_v.at[1 - hh], sem_h)

            _issue_rows(hh, 0, 0)

            @pl.loop(0, cc // 2)
            def _bpair(j):
                for par in range(2):
                    r = 2 * j + par
                    bl = ci * cc + r

                    @pl.when(r < cc - 1)
                    def _():
                        _issue_rows(hh, r + 1, 1 - par)

                    _drain_scores(hh, r)
                    den = jnp.zeros((16,), jnp.float32)
                    for k in range(LP // 16):
                        e = jnp.exp(s2_v[hh, r, pl.ds(16 * k, 16)])
                        w_v[pl.ds(16 * k, 16)] = e
                        den = den + e
                    _drain_rows(par)

                    z = jnp.zeros((16,), jnp.float32)

                    @pl.loop(0, LP // 16, init_carry=(z,) * 8)
                    def _acc(k, carry):
                        wv = w_v[pl.ds(16 * k, 16)]
                        carry = list(carry)
                        for jj in range(16):
                            wl = wv[jj]
                            l = 16 * k + jj
                            h = (jj & 1) * 4
                            for q in range(4):
                                carry[h + q] = (carry[h + q] +
                                                rows_v[par, l,
                                                       pl.ds(16 * q, 16)] * wl)
                        return tuple(carry)

                    ac = _acc
                    out_v[bl, pl.ds(0, 16)] = ac[0] + ac[4]
                    out_v[bl, pl.ds(16, 16)] = ac[1] + ac[5]
                    out_v[bl, pl.ds(32, 16)] = ac[2] + ac[6]
                    out_v[bl, pl.ds(48, 16)] = ac[3] + ac[7]
                    den_v[bl, :] = den

            @pl.when(ci + 1 < nchunk)
            def _():
                pltpu.make_async_copy(hist_hbm.at[pl.ds(0, cc), :],
                                      hist_v.at[1 - hh], sem_h).wait()
                for r in range(cc):
                    _issue_scores(1 - hh, r)

    pltpu.sync_copy(out_v, hnum_hbm.at[pl.ds(b0, bpw)])
    pltpu.sync_copy(den_v, den_hbm.at[pl.ds(b0, bpw)])
    for c in id_descs:
        c.wait()
    pltpu.sync_copy(ie_v, iemb_hbm.at[pl.ds(b0, bpw)])


def _sc_gather(history, g, news_table, news_idx, user_idx, user_table):
    bpw = B // 32
    mesh = plsc.VectorSubcoreMesh(core_axis_name="c", subcore_axis_name="s")
    f = pl.kernel(
        _sc_body,
        out_type=(
            jax.ShapeDtypeStruct((B, D), jnp.float32),
            jax.ShapeDtypeStruct((B, 16), jnp.float32),
            jax.ShapeDtypeStruct((B, D), jnp.float32),
            jax.ShapeDtypeStruct((B, D), jnp.float32),
        ),
        mesh=mesh,
        scratch_types=[
            pltpu.VMEM((2, 16, L), jnp.int32),
            pltpu.VMEM((2, 16, LP), jnp.float32),
            pltpu.VMEM((LP,), jnp.float32),
            pltpu.VMEM((2, LP, D), jnp.float32),
            pltpu.VMEM((bpw, D), jnp.float32),
            pltpu.VMEM((bpw, 16), jnp.float32),
            pltpu.VMEM((bpw, D), jnp.float32),
            pltpu.VMEM((bpw,), jnp.int32),
            pltpu.SemaphoreType.DMA,
            pltpu.SemaphoreType.DMA,
            pltpu.SemaphoreType.DMA,
            pltpu.SemaphoreType.DMA,
            pltpu.SemaphoreType.DMA,
            pltpu.SemaphoreType.DMA,
        ],
        compiler_params=pltpu.CompilerParams(use_tc_tiling_on_sc=False),
    )
    return f(history, g, news_table, news_idx, user_idx, user_table)


_RB = 2048


def _final_body(ue_ref, hn_ref, den_ref, ie_ref, wut_ref, but_ref, wnt_ref,
                bnt_ref, out_ref):
    den = jnp.sum(den_ref[...], axis=1, keepdims=True)
    hr = hn_ref[...] * jnp.where(den > 0, 1.0 / den, 0.0)
    u = ue_ref[...] + hr
    ur = jax.nn.relu(
        lax.dot_general(u, wut_ref[...], (((1,), (1,)), ((), ())),
                        preferred_element_type=jnp.float32)
        + but_ref[...][None, :])
    nr = jax.nn.relu(
        lax.dot_general(ie_ref[...], wnt_ref[...], (((1,), (1,)), ((), ())),
                        preferred_element_type=jnp.float32)
        + bnt_ref[...][None, :])
    out_ref[...] = jax.nn.sigmoid(jnp.sum(ur * nr, axis=1))


def _final(user_emb, hist_num, den, id_emb, W_ut, b_ut, W_nt, b_nt):
    grid = B // _RB
    return pl.pallas_call(
        _final_body,
        grid=(grid,),
        in_specs=[
            pl.BlockSpec((_RB, D), lambda i: (i, 0)),
            pl.BlockSpec((_RB, D), lambda i: (i, 0)),
            pl.BlockSpec((_RB, 16), lambda i: (i, 0)),
            pl.BlockSpec((_RB, D), lambda i: (i, 0)),
            pl.BlockSpec((D, D), lambda i: (0, 0)),
            pl.BlockSpec((D,), lambda i: (0,)),
            pl.BlockSpec((D, D), lambda i: (0, 0)),
            pl.BlockSpec((D,), lambda i: (0,)),
        ],
        out_specs=pl.BlockSpec((_RB,), lambda i: (i,)),
        out_shape=jax.ShapeDtypeStruct((B,), jnp.float32),
    )(user_emb, hist_num, den, id_emb, W_ut, b_ut, W_nt, b_nt)


def kernel(user_idx, news_idx, history, user_table, news_table,
           W_ut, b_ut, W_nt, b_nt, W_a1, b_a1, W_a2, b_a2):
    news_t = news_table.T
    g = _score_table(news_t, W_a1, b_a1, W_a2, b_a2)
    hist_num, den, id_emb, user_emb = _sc_gather(
        history, g, news_table, news_idx, user_idx, user_table)
    return _final(user_emb, hist_num, den, id_emb, W_ut, b_ut, W_nt, b_nt)


# speedup vs baseline: 1.1922x; 1.1922x over previous
import jax
import jax.numpy as jnp
from jax import lax
from jax.experimental import pallas as pl
from jax.experimental.pallas import tpu as pltpu
from jax.experimental.pallas import tpu_sc as plsc

B = 16384
L = 200
D = 64
N = 1_000_000
LP = 208
NEG = -1e9

_BK = 8192


def _score_table_body(ntt_ref, w1_ref, b1_ref, w2_ref, b2_ref, out_ref):
    i = pl.program_id(0)
    x = ntt_ref[...]
    a = jnp.tanh(
        lax.dot_general(w1_ref[...], x, (((1,), (0,)), ((), ())),
                        preferred_element_type=jnp.float32)
        + b1_ref[...][:, None])
    g = lax.dot_general(w2_ref[...], a, (((1,), (0,)), ((), ())),
                        preferred_element_type=jnp.float32)
    g = (g + b2_ref[...][None, :]).reshape(_BK)
    nidx = lax.broadcasted_iota(jnp.int32, (_BK,), 0) + i * _BK
    out_ref[...] = jnp.where(nidx == 0, jnp.float32(NEG), g)


def _score_table(news_t, W_a1, b_a1, W_a2, b_a2):
    grid = pl.cdiv(N, _BK)
    return pl.pallas_call(
        _score_table_body,
        grid=(grid,),
        in_specs=[
            pl.BlockSpec((D, _BK), lambda i: (0, i)),
            pl.BlockSpec((D // 2, D), lambda i: (0, 0)),
            pl.BlockSpec((D // 2,), lambda i: (0,)),
            pl.BlockSpec((1, D // 2), lambda i: (0, 0)),
            pl.BlockSpec((1,), lambda i: (0,)),
        ],
        out_specs=pl.BlockSpec((_BK,), lambda i: (i,)),
        out_shape=jax.ShapeDtypeStruct((N,), jnp.float32),
    )(news_t, W_a1, b_a1, W_a2, b_a2)


def _sc_body(hist_hbm, g_hbm, news_hbm, nidx_hbm,
             hnum_hbm, den_hbm, iemb_hbm,
             hist_v, s2_v, w_v, rows_v, out_v, den_v, ie_v, idx_v,
             sem_s0, sem_s1, sem_r0, sem_r1, sem_h, sem_i):
    nc = 2
    wid = lax.axis_index("s") * nc + lax.axis_index("c")
    bpw = B // 32
    b0 = wid * bpw
    cc = 16
    nchunk = bpw // cc
    sem_s = (sem_s0, sem_s1)
    sem_r = (sem_r0, sem_r1)

    zf = jnp.zeros((16,), jnp.float32)
    for hh in range(2):
        for r in range(cc):
            s2_v[hh, r, pl.ds(192, 16)] = jnp.full((16,), NEG, jnp.float32)
    for nb in range(2):
        for r in range(L, LP):
            for q in range(D // 16):
                rows_v[nb, r, pl.ds(16 * q, 16)] = zf

    pltpu.sync_copy(nidx_hbm.at[pl.ds(b0, bpw)], idx_v)
    id_descs = [
        pltpu.async_copy(news_hbm.at[idx_v.at[pl.ds(j * 128, 128)]],
                         ie_v.at[pl.ds(j * 128, 128), :], sem_i)
        for j in range(bpw // 128)
    ]

    def _issue_scores(hh, r):
        pltpu.async_copy(g_hbm.at[hist_v.at[hh, r, pl.ds(0, 104)]],
                         s2_v.at[hh, r, pl.ds(0, 104)], sem_s[hh])
        pltpu.async_copy(g_hbm.at[hist_v.at[hh, r, pl.ds(104, 96)]],
                         s2_v.at[hh, r, pl.ds(104, 96)], sem_s[hh])

    def _drain_scores(hh, r):
        pltpu.make_async_copy(g_hbm.at[pl.ds(0, 104)],
                              s2_v.at[hh, r, pl.ds(0, 104)], sem_s[hh]).wait()
        pltpu.make_async_copy(g_hbm.at[pl.ds(0, 96)],
                              s2_v.at[hh, r, pl.ds(104, 96)], sem_s[hh]).wait()

    def _issue_rows(hh, r, nb):
        pltpu.async_copy(news_hbm.at[hist_v.at[hh, r, pl.ds(0, 104)]],
                         rows_v.at[nb, pl.ds(0, 104), :], sem_r[nb])
        pltpu.async_copy(news_hbm.at[hist_v.at[hh, r, pl.ds(104, 96)]],
                         rows_v.at[nb, pl.ds(104, 96), :], sem_r[nb])

    def _drain_rows(nb):
        pltpu.make_async_copy(news_hbm.at[pl.ds(0, 104), :],
                              rows_v.at[nb, pl.ds(0, 104), :],
                              sem_r[nb]).wait()
        pltpu.make_async_copy(news_hbm.at[pl.ds(0, 96), :],
                              rows_v.at[nb, pl.ds(104, 96), :],
                              sem_r[nb]).wait()

    pltpu.async_copy(hist_hbm.at[pl.ds(b0, cc), :], hist_v.at[0],
                     sem_h).wait()
    for r in range(cc):
        _issue_scores(0, r)

    @pl.loop(0, nchunk // 2)
    def _pair(i):
        for hh in range(2):
            ci = 2 * i + hh

            @pl.when(ci + 1 < nchunk)
            def _():
                pltpu.async_copy(
                    hist_hbm.at[pl.ds(b0 + (ci + 1) * cc, cc), :],
                    hist_v.at[1 - hh], sem_h)

            _issue_rows(hh, 0, 0)

            @pl.loop(0, cc // 2)
            def _bpair(j):
                for par in range(2):
                    r = 2 * j + par
                    bl = ci * cc + r

                    @pl.when(r < cc - 1)
                    def _():
                        _issue_rows(hh, r + 1, 1 - par)

                    _drain_scores(hh, r)
                    den = jnp.zeros((16,), jnp.float32)
                    for k in range(LP // 16):
                        e = jnp.exp(s2_v[hh, r, pl.ds(16 * k, 16)])
                        w_v[pl.ds(16 * k, 16)] = e
                        den = den + e
                    _drain_rows(par)

                    z = jnp.zeros((16,), jnp.float32)

                    @pl.loop(0, LP // 16, init_carry=(z,) * 8)
                    def _acc(k, carry):
                        wv = w_v[pl.ds(16 * k, 16)]
                        carry = list(carry)
                        for jj in range(16):
                            wl = wv[jj]
                            l = 16 * k + jj
                            h = (jj & 1) * 4
                            for q in range(4):
                                carry[h + q] = (carry[h + q] +
                                                rows_v[par, l,
                                                       pl.ds(16 * q, 16)] * wl)
                        return tuple(carry)

                    ac = _acc
                    out_v[bl, pl.ds(0, 16)] = ac[0] + ac[4]
                    out_v[bl, pl.ds(16, 16)] = ac[1] + ac[5]
                    out_v[bl, pl.ds(32, 16)] = ac[2] + ac[6]
                    out_v[bl, pl.ds(48, 16)] = ac[3] + ac[7]
                    den_v[bl, :] = den

            @pl.when(ci + 1 < nchunk)
            def _():
                pltpu.make_async_copy(hist_hbm.at[pl.ds(0, cc), :],
                                      hist_v.at[1 - hh], sem_h).wait()
                for r in range(cc):
                    _issue_scores(1 - hh, r)

    pltpu.sync_copy(out_v, hnum_hbm.at[pl.ds(b0, bpw)])
    pltpu.sync_copy(den_v, den_hbm.at[pl.ds(b0, bpw)])
    for c in id_descs:
        c.wait()
    pltpu.sync_copy(ie_v, iemb_hbm.at[pl.ds(b0, bpw)])


def _sc_gather(history, g, news_table, news_idx):
    bpw = B // 32
    mesh = plsc.VectorSubcoreMesh(core_axis_name="c", subcore_axis_name="s")
    f = pl.kernel(
        _sc_body,
        out_type=(
            jax.ShapeDtypeStruct((B, D), jnp.float32),
            jax.ShapeDtypeStruct((B, 16), jnp.float32),
            jax.ShapeDtypeStruct((B, D), jnp.float32),
        ),
        mesh=mesh,
        scratch_types=[
            pltpu.VMEM((2, 16, L), jnp.int32),
            pltpu.VMEM((2, 16, LP), jnp.float32),
            pltpu.VMEM((LP,), jnp.float32),
            pltpu.VMEM((2, LP, D), jnp.float32),
            pltpu.VMEM((bpw, D), jnp.float32),
            pltpu.VMEM((bpw, 16), jnp.float32),
            pltpu.VMEM((bpw, D), jnp.float32),
            pltpu.VMEM((bpw,), jnp.int32),
            pltpu.SemaphoreType.DMA,
            pltpu.SemaphoreType.DMA,
            pltpu.SemaphoreType.DMA,
            pltpu.SemaphoreType.DMA,
            pltpu.SemaphoreType.DMA,
            pltpu.SemaphoreType.DMA,
        ],
        compiler_params=pltpu.CompilerParams(use_tc_tiling_on_sc=False),
    )
    return f(history, g, news_table, news_idx)


def _user_body(uidx_hbm, utab_hbm, uemb_hbm, out_v, idx_v, sem):
    nc = 2
    wid = lax.axis_index("s") * nc + lax.axis_index("c")
    bpw = B // 32
    b0 = wid * bpw
    pltpu.sync_copy(uidx_hbm.at[pl.ds(b0, bpw)], idx_v)
    descs = [
        pltpu.async_copy(utab_hbm.at[idx_v.at[pl.ds(j * 128, 128)]],
                         out_v.at[pl.ds(j * 128, 128), :], sem)
        for j in range(bpw // 128)
    ]
    for c in descs:
        c.wait()
    pltpu.sync_copy(out_v, uemb_hbm.at[pl.ds(b0, bpw)])


def _user_gather(user_idx, user_table):
    bpw = B // 32
    mesh = plsc.VectorSubcoreMesh(core_axis_name="c", subcore_axis_name="s")
    f = pl.kernel(
        _user_body,
        out_type=jax.ShapeDtypeStruct((B, D), jnp.float32),
        mesh=mesh,
        scratch_types=[
            pltpu.VMEM((bpw, D), jnp.float32),
            pltpu.VMEM((bpw,), jnp.int32),
            pltpu.SemaphoreType.DMA,
        ],
        compiler_params=pltpu.CompilerParams(use_tc_tiling_on_sc=False),
    )
    return f(user_idx, user_table)


_RB = 2048


def _final_body(ue_ref, hn_ref, den_ref, ie_ref, wut_ref, but_ref, wnt_ref,
                bnt_ref, out_ref):
    den = jnp.sum(den_ref[...], axis=1, keepdims=True)
    hr = hn_ref[...] * jnp.where(den > 0, 1.0 / den, 0.0)
    u = ue_ref[...] + hr
    ur = jax.nn.relu(
        lax.dot_general(u, wut_ref[...], (((1,), (1,)), ((), ())),
                        preferred_element_type=jnp.float32)
        + but_ref[...][None, :])
    nr = jax.nn.relu(
        lax.dot_general(ie_ref[...], wnt_ref[...], (((1,), (1,)), ((), ())),
                        preferred_element_type=jnp.float32)
        + bnt_ref[...][None, :])
    out_ref[...] = jax.nn.sigmoid(jnp.sum(ur * nr, axis=1))


def _final(user_emb, hist_num, den, id_emb, W_ut, b_ut, W_nt, b_nt):
    grid = B // _RB
    return pl.pallas_call(
        _final_body,
        grid=(grid,),
        in_specs=[
            pl.BlockSpec((_RB, D), lambda i: (i, 0)),
            pl.BlockSpec((_RB, D), lambda i: (i, 0)),
            pl.BlockSpec((_RB, 16), lambda i: (i, 0)),
            pl.BlockSpec((_RB, D), lambda i: (i, 0)),
            pl.BlockSpec((D, D), lambda i: (0, 0)),
            pl.BlockSpec((D,), lambda i: (0,)),
            pl.BlockSpec((D, D), lambda i: (0, 0)),
            pl.BlockSpec((D,), lambda i: (0,)),
        ],
        out_specs=pl.BlockSpec((_RB,), lambda i: (i,)),
        out_shape=jax.ShapeDtypeStruct((B,), jnp.float32),
    )(user_emb, hist_num, den, id_emb, W_ut, b_ut, W_nt, b_nt)


def kernel(user_idx, news_idx, history, user_table, news_table,
           W_ut, b_ut, W_nt, b_nt, W_a1, b_a1, W_a2, b_a2):
    news_t = news_table.T
    g = _score_table(news_t, W_a1, b_a1, W_a2, b_a2)
    hist_num, den, id_emb = _sc_gather(history, g, news_table, news_idx)
    user_emb = _user_gather(user_idx, user_table)
    return _final(user_emb, hist_num, den, id_emb, W_ut, b_ut, W_nt, b_nt)


# speedup vs baseline: 1.5999x; 1.3419x over previous
import jax
import jax.numpy as jnp
from jax import lax
from jax.experimental import pallas as pl
from jax.experimental.pallas import tpu as pltpu
from jax.experimental.pallas import tpu_sc as plsc

B = 16384
L = 200
D = 64
N = 1_000_000
LP = 208
NEG = -1e9

_BK = 4096
HALF = 524288


def _score_table_body(x1_ref, x2_ref, w1_ref, b1_ref, w2_ref, b2_ref,
                      o128_ref, g1_ref, g2_ref):
    i = pl.program_id(0)
    x1 = x1_ref[...]
    x2 = x2_ref[...]
    o128_ref[...] = jnp.concatenate(
        [jnp.swapaxes(x1, 0, 1), jnp.swapaxes(x2, 0, 1)], axis=1)

    def scores(x):
        a = jnp.tanh(
            lax.dot_general(w1_ref[...], x, (((1,), (0,)), ((), ())),
                            preferred_element_type=jnp.float32)
            + b1_ref[...][:, None])
        g = lax.dot_general(w2_ref[...], a, (((1,), (0,)), ((), ())),
                            preferred_element_type=jnp.float32)
        return (g + b2_ref[...][:, None]).reshape(_BK)

    g1 = scores(x1)
    nidx = lax.broadcasted_iota(jnp.int32, (_BK,), 0) + i * _BK
    g1_ref[...] = jnp.where(nidx == 0, jnp.float32(NEG), g1)
    g2_ref[...] = scores(x2)


def _score_table(news_t, W_a1, b_a1, W_a2, b_a2):
    grid = HALF // _BK
    return pl.pallas_call(
        _score_table_body,
        grid=(grid,),
        in_specs=[
            pl.BlockSpec((D, _BK), lambda i: (0, i)),
            pl.BlockSpec((D, _BK),
                         lambda i: (0, jnp.minimum(i + HALF // _BK,
                                                   N // _BK))),
            pl.BlockSpec((D // 2, D), lambda i: (0, 0)),
            pl.BlockSpec((D // 2,), lambda i: (0,)),
            pl.BlockSpec((1, D // 2), lambda i: (0, 0)),
            pl.BlockSpec((1,), lambda i: (0,)),
        ],
        out_specs=(
            pl.BlockSpec((_BK, 128), lambda i: (i, 0)),
            pl.BlockSpec((_BK,), lambda i: (i,)),
            pl.BlockSpec((_BK,), lambda i: (i,)),
        ),
        out_shape=(
            jax.ShapeDtypeStruct((HALF, 128), jnp.float32),
            jax.ShapeDtypeStruct((HALF,), jnp.float32),
            jax.ShapeDtypeStruct((HALF,), jnp.float32),
        ),
    )(news_t, news_t, W_a1, b_a1, W_a2, b_a2)


def _sc_body(hist_hbm, g_hbm, news_hbm, nidx_hbm,
             hnum_hbm, den_hbm, iemb_hbm,
             hist_v, hist2_v, s2_v, w_v, rows_v, out_v, den_v, ie_v, idx_v,
             sem_s0, sem_s1, sem_r0, sem_r1, sem_h, sem_i):
    nc = 2
    wid = lax.axis_index("s") * nc + lax.axis_index("c")
    bpw = B // 32
    b0 = wid * bpw
    cc = 16
    nchunk = bpw // cc
    sem_s = (sem_s0, sem_s1)
    sem_r = (sem_r0, sem_r1)

    zf = jnp.zeros((16,), jnp.float32)
    for hh in range(2):
        for r in range(cc):
            s2_v[hh, r, pl.ds(192, 16)] = jnp.full((16,), NEG, jnp.float32)
    for nb in range(2):
        for r in range(L, LP):
            for q in range(D // 16):
                rows_v[nb, r, pl.ds(16 * q, 16)] = zf

    def _xf(v):
        return (v << 1) - jnp.where(v < HALF, 0, 2 * HALF - 1)

    def _xform_chunk(hh):
        @pl.loop(0, cc)
        def _row(r):
            for k in range(13):
                off = min(16 * k, L - 16)
                hist2_v[hh, r, pl.ds(off, 16)] = _xf(
                    hist_v[hh, r, pl.ds(off, 16)])

    pltpu.sync_copy(nidx_hbm.at[pl.ds(b0, bpw)], idx_v)
    for j in range(bpw // 16):
        idx_v[pl.ds(16 * j, 16)] = _xf(idx_v[pl.ds(16 * j, 16)])
    id_descs = [
        pltpu.async_copy(news_hbm.at[idx_v.at[pl.ds(j * 128, 128)]],
                         ie_v.at[pl.ds(j * 128, 128), :], sem_i)
        for j in range(bpw // 128)
    ]

    def _issue_scores(hh, r):
        pltpu.async_copy(g_hbm.at[hist_v.at[hh, r, pl.ds(0, 104)]],
                         s2_v.at[hh, r, pl.ds(0, 104)], sem_s[hh])
        pltpu.async_copy(g_hbm.at[hist_v.at[hh, r, pl.ds(104, 96)]],
                         s2_v.at[hh, r, pl.ds(104, 96)], sem_s[hh])

    def _drain_scores(hh, r):
        pltpu.make_async_copy(g_hbm.at[pl.ds(0, 104)],
                              s2_v.at[hh, r, pl.ds(0, 104)], sem_s[hh]).wait()
        pltpu.make_async_copy(g_hbm.at[pl.ds(0, 96)],
                              s2_v.at[hh, r, pl.ds(104, 96)], sem_s[hh]).wait()

    def _issue_rows(hh, r, nb):
        pltpu.async_copy(news_hbm.at[hist2_v.at[hh, r, pl.ds(0, 104)]],
                         rows_v.at[nb, pl.ds(0, 104), :], sem_r[nb])
        pltpu.async_copy(news_hbm.at[hist2_v.at[hh, r, pl.ds(104, 96)]],
                         rows_v.at[nb, pl.ds(104, 96), :], sem_r[nb])

    def _drain_rows(nb):
        pltpu.make_async_copy(news_hbm.at[pl.ds(0, 104), :],
                              rows_v.at[nb, pl.ds(0, 104), :],
                              sem_r[nb]).wait()
        pltpu.make_async_copy(news_hbm.at[pl.ds(0, 96), :],
                              rows_v.at[nb, pl.ds(104, 96), :],
                              sem_r[nb]).wait()

    pltpu.async_copy(hist_hbm.at[pl.ds(b0, cc), :], hist_v.at[0],
                     sem_h).wait()
    for r in range(cc):
        _issue_scores(0, r)
    _xform_chunk(0)

    @pl.loop(0, nchunk // 2)
    def _pair(i):
        for hh in range(2):
            ci = 2 * i + hh

            @pl.when(ci + 1 < nchunk)
            def _():
                pltpu.async_copy(
                    hist_hbm.at[pl.ds(b0 + (ci + 1) * cc, cc), :],
                    hist_v.at[1 - hh], sem_h)

            _issue_rows(hh, 0, 0)

            @pl.loop(0, cc // 2)
            def _bpair(j):
                for par in range(2):
                    r = 2 * j + par
                    bl = ci * cc + r

                    @pl.when(r < cc - 1)
                    def _():
                        _issue_rows(hh, r + 1, 1 - par)

                    _drain_scores(hh, r)
                    den = jnp.zeros((16,), jnp.float32)
                    for k in range(LP // 16):
                        e = jnp.exp(s2_v[hh, r, pl.ds(16 * k, 16)])
                        w_v[pl.ds(16 * k, 16)] = e
                        den = den + e
                    _drain_rows(par)

                    z = jnp.zeros((16,), jnp.float32)

                    @pl.loop(0, LP // 16, init_carry=(z,) * 8)
                    def _acc(k, carry):
                        wv = w_v[pl.ds(16 * k, 16)]
                        carry = list(carry)
                        for jj in range(16):
                            wl = wv[jj]
                            l = 16 * k + jj
                            h = (jj & 1) * 4
                            for q in range(4):
                                carry[h + q] = (carry[h + q] +
                                                rows_v[par, l,
                                                       pl.ds(16 * q, 16)] * wl)
                        return tuple(carry)

                    ac = _acc
                    out_v[bl, pl.ds(0, 16)] = ac[0] + ac[4]
                    out_v[bl, pl.ds(16, 16)] = ac[1] + ac[5]
                    out_v[bl, pl.ds(32, 16)] = ac[2] + ac[6]
                    out_v[bl, pl.ds(48, 16)] = ac[3] + ac[7]
                    den_v[bl, :] = den

            @pl.when(ci + 1 < nchunk)
            def _():
                pltpu.make_async_copy(hist_hbm.at[pl.ds(0, cc), :],
                                      hist_v.at[1 - hh], sem_h).wait()
                for r in range(cc):
                    _issue_scores(1 - hh, r)
                _xform_chunk(1 - hh)

    pltpu.sync_copy(out_v, hnum_hbm.at[pl.ds(b0, bpw)])
    pltpu.sync_copy(den_v, den_hbm.at[pl.ds(b0, bpw)])
    for c in id_descs:
        c.wait()
    pltpu.sync_copy(ie_v, iemb_hbm.at[pl.ds(b0, bpw)])


def _sc_gather(history, g, news_table, news_idx):
    bpw = B // 32
    mesh = plsc.VectorSubcoreMesh(core_axis_name="c", subcore_axis_name="s")
    f = pl.kernel(
        _sc_body,
        out_type=(
            jax.ShapeDtypeStruct((B, D), jnp.float32),
            jax.ShapeDtypeStruct((B, 16), jnp.float32),
            jax.ShapeDtypeStruct((B, D), jnp.float32),
        ),
        mesh=mesh,
        scratch_types=[
            pltpu.VMEM((2, 16, L), jnp.int32),
            pltpu.VMEM((2, 16, L), jnp.int32),
            pltpu.VMEM((2, 16, LP), jnp.float32),
            pltpu.VMEM((LP,), jnp.float32),
            pltpu.VMEM((2, LP, D), jnp.float32),
            pltpu.VMEM((bpw, D), jnp.float32),
            pltpu.VMEM((bpw, 16), jnp.float32),
            pltpu.VMEM((bpw, D), jnp.float32),
            pltpu.VMEM((bpw,), jnp.int32),
            pltpu.SemaphoreType.DMA,
            pltpu.SemaphoreType.DMA,
            pltpu.SemaphoreType.DMA,
            pltpu.SemaphoreType.DMA,
            pltpu.SemaphoreType.DMA,
            pltpu.SemaphoreType.DMA,
        ],
        compiler_params=pltpu.CompilerParams(use_tc_tiling_on_sc=False),
    )
    return f(history, g, news_table, news_idx)


def _user_body(uidx_hbm, utab_hbm, uemb_hbm, out_v, idx_v, sem):
    nc = 2
    wid = lax.axis_index("s") * nc + lax.axis_index("c")
    bpw = B // 32
    b0 = wid * bpw
    pltpu.sync_copy(uidx_hbm.at[pl.ds(b0, bpw)], idx_v)
    descs = [
        pltpu.async_copy(utab_hbm.at[idx_v.at[pl.ds(j * 128, 128)]],
                         out_v.at[pl.ds(j * 128, 128), :], sem)
        for j in range(bpw // 128)
    ]
    for c in descs:
        c.wait()
    pltpu.sync_copy(out_v, uemb_hbm.at[pl.ds(b0, bpw)])


def _user_gather(user_idx, user_table):
    bpw = B // 32
    mesh = plsc.VectorSubcoreMesh(core_axis_name="c", subcore_axis_name="s")
    f = pl.kernel(
        _user_body,
        out_type=jax.ShapeDtypeStruct((B, D), jnp.float32),
        mesh=mesh,
        scratch_types=[
            pltpu.VMEM((bpw, D), jnp.float32),
            pltpu.VMEM((bpw,), jnp.int32),
            pltpu.SemaphoreType.DMA,
        ],
        compiler_params=pltpu.CompilerParams(use_tc_tiling_on_sc=False),
    )
    return f(user_idx, user_table)


_RB = 2048


def _final_body(ue_ref, hn_ref, den_ref, ie_ref, wut_ref, but_ref, wnt_ref,
                bnt_ref, out_ref):
    den = jnp.sum(den_ref[...], axis=1, keepdims=True)
    hr = hn_ref[...] * jnp.where(den > 0, 1.0 / den, 0.0)
    u = ue_ref[...] + hr
    ur = jax.nn.relu(
        lax.dot_general(u, wut_ref[...], (((1,), (1,)), ((), ())),
                        preferred_element_type=jnp.float32)
        + but_ref[...][None, :])
    nr = jax.nn.relu(
        lax.dot_general(ie_ref[...], wnt_ref[...], (((1,), (1,)), ((), ())),
                        preferred_element_type=jnp.float32)
        + bnt_ref[...][None, :])
    out_ref[...] = jax.nn.sigmoid(jnp.sum(ur * nr, axis=1))


def _final(user_emb, hist_num, den, id_emb, W_ut, b_ut, W_nt, b_nt):
    grid = B // _RB
    return pl.pallas_call(
        _final_body,
        grid=(grid,),
        in_specs=[
            pl.BlockSpec((_RB, D), lambda i: (i, 0)),
            pl.BlockSpec((_RB, D), lambda i: (i, 0)),
            pl.BlockSpec((_RB, 16), lambda i: (i, 0)),
            pl.BlockSpec((_RB, D), lambda i: (i, 0)),
            pl.BlockSpec((D, D), lambda i: (0, 0)),
            pl.BlockSpec((D,), lambda i: (0,)),
            pl.BlockSpec((D, D), lambda i: (0, 0)),
            pl.BlockSpec((D,), lambda i: (0,)),
        ],
        out_specs=pl.BlockSpec((_RB,), lambda i: (i,)),
        out_shape=jax.ShapeDtypeStruct((B,), jnp.float32),
    )(user_emb, hist_num, den, id_emb, W_ut, b_ut, W_nt, b_nt)


def kernel(user_idx, news_idx, history, user_table, news_table,
           W_ut, b_ut, W_nt, b_nt, W_a1, b_a1, W_a2, b_a2):
    news_t = news_table.T
    nt128, g1, g2 = _score_table(news_t, W_a1, b_a1, W_a2, b_a2)
    g = jnp.concatenate([g1, g2])
    news64 = nt128.reshape(2 * HALF, D)
    hist_num, den, id_emb = _sc_gather(history, g, news64, news_idx)
    user_emb = _user_gather(user_idx, user_table)
    return _final(user_emb, hist_num, den, id_emb, W_ut, b_ut, W_nt, b_nt)
